# baseline (speedup 1.0000x reference)
import math

import jax
import jax.numpy as jnp
from jax.experimental import pallas as pl
from jax.experimental.pallas import tpu as pltpu

B = 16
Z = 4096
D = 768
O = 32
_SCALE = 1.0 / math.sqrt(D)


def _zoner_kernel(txt_ref, zone_ref, wt_ref, bt_ref, wz_ref, bz_ref,
                  mask_ref, out_ref):
    b = pl.program_id(0)
    t = jnp.tanh(
        jnp.dot(txt_ref[pl.ds(b, 1), :], wt_ref[...],
                preferred_element_type=jnp.float32) + bt_ref[...])
    z = jnp.tanh(
        jnp.dot(zone_ref[0], wz_ref[...],
                preferred_element_type=jnp.float32) + bz_ref[...])
    logits = jax.lax.dot_general(
        t, z, (((1,), (1,)), ((), ())),
        preferred_element_type=jnp.float32) * _SCALE
    logits = jnp.where(mask_ref[0], -jnp.inf, logits)
    m = jnp.max(logits, axis=1, keepdims=True)
    e = jnp.exp(logits - m)
    out_ref[0] = e / jnp.sum(e, axis=1, keepdims=True)


def kernel(txt_embeds, zone_embeds, W_txt, b_txt, W_zone, b_zone, mask):
    wt = W_txt.T
    wz = W_zone.T
    bt = b_txt.reshape(1, O)
    bz = b_zone.reshape(1, O)

    mask3 = mask.reshape(B, 1, Z)

    out = pl.pallas_call(
        _zoner_kernel,
        grid=(B,),
        in_specs=[
            pl.BlockSpec((B, D), lambda b: (0, 0)),
            pl.BlockSpec((1, Z, D), lambda b: (b, 0, 0)),
            pl.BlockSpec((D, O), lambda b: (0, 0)),
            pl.BlockSpec((1, O), lambda b: (0, 0)),
            pl.BlockSpec((D, O), lambda b: (0, 0)),
            pl.BlockSpec((1, O), lambda b: (0, 0)),
            pl.BlockSpec((1, 1, Z), lambda b: (b, 0, 0)),
        ],
        out_specs=pl.BlockSpec((1, 1, Z), lambda b: (b, 0, 0)),
        out_shape=jax.ShapeDtypeStruct((B, 1, Z), jnp.float32),
    )(txt_embeds, zone_embeds, wt, bt, wz, bz, mask3)
    return out.reshape(B, Z)
